# baseline (speedup 1.0000x reference)
import functools

import jax
import jax.numpy as jnp
from jax import lax
from jax.experimental import pallas as pl
from jax.experimental.pallas import tpu as pltpu
from jax.experimental.pallas import tpu_sc as plsc

N_NODES = 10000
N_EDGES = 320000
D = 128

NC = 2
NS = 16
NW = NC * NS
CH = 128
NCH = 80
EPT = NCH * CH
E_PAD = NW * EPT
ACC_ROWS = 10240
RPT = ACC_ROWS // NS
TRASH_ROW = N_NODES + 7
ROW_BLK = 400
TOT_CH = E_PAD // CH
C0_CH = 160
C1_CH = TOT_CH // NS - C0_CH


def _sc_mesh():
  return plsc.VectorSubcoreMesh(core_axis_name="c", subcore_axis_name="s",
                                num_cores=NC, num_subcores=NS)


@functools.lru_cache(maxsize=None)
def _make_sc_segsum():
  out_type = jax.ShapeDtypeStruct((NC, ACC_ROWS, D), jnp.float32)

  NBUF = 2
  SR = 8
  scratch_types = [
      pltpu.VMEM((SR, CH), jnp.int32),
      pltpu.VMEM((SR, CH), jnp.int32),
      pltpu.VMEM((NBUF * CH, D), jnp.float32),
      pltpu.VMEM_SHARED((ACC_ROWS, D), jnp.float32),
      pltpu.SemaphoreType.DMA((NBUF,)),
      pltpu.SemaphoreType.DMA((SR,)),
      pltpu.SemaphoreType.DMA((SR,)),
  ]

  def body(u_hbm, src_hbm, dst_hbm, zrow_hbm, out_s,
           src_r, dst_r, rows_v, acc_sh, gsem, ssem, dsem):
    rows = [rows_v.at[pl.ds(p * CH, CH)] for p in range(NBUF)]
    cid = lax.axis_index("c")
    sid = lax.axis_index("s")
    base = sid * RPT

    pltpu.sync_copy(zrow_hbm, acc_sh.at[pl.ds(base, RPT)])
    plsc.subcore_barrier()

    def run(nch, ck0):
      for r in range(SR):
        pltpu.async_copy(src_hbm.at[ck0 + r], src_r.at[r], ssem.at[r])
        pltpu.async_copy(dst_hbm.at[ck0 + r], dst_r.at[r], dsem.at[r])
      for p in range(NBUF):
        pltpu.make_async_copy(src_hbm.at[0], src_r.at[p], ssem.at[p]).wait()
        pltpu.async_copy(u_hbm.at[src_r.at[p]], rows[p], gsem.at[p])

      def step(j, b, p, pre_idx, pre_gather):
        pltpu.make_async_copy(u_hbm.at[pl.ds(0, CH)], rows[p],
                              gsem.at[p]).wait()
        pltpu.make_async_copy(dst_hbm.at[0], dst_r.at[b], dsem.at[b]).wait()
        pltpu.sync_copy(rows[p], acc_sh.at[dst_r.at[b]], add=True)
        if pre_idx:
          pltpu.async_copy(src_hbm.at[ck0 + j + SR], src_r.at[b], ssem.at[b])
          pltpu.async_copy(dst_hbm.at[ck0 + j + SR], dst_r.at[b], dsem.at[b])
        if pre_gather:
          b2 = b + NBUF if b + NBUF < SR else b + NBUF - SR
          pltpu.make_async_copy(src_hbm.at[0], src_r.at[b2],
                                ssem.at[b2]).wait()
          pltpu.async_copy(u_hbm.at[src_r.at[b2]], rows[p], gsem.at[p])

      @pl.loop(0, nch - SR, step=SR)
      def _chunk(j0):
        for b in range(SR):
          step(j0 + b, b, b % NBUF, True, True)

      for jj in range(SR):
        j = nch - SR + jj
        step(j, jj % SR, jj % NBUF, False, jj + NBUF < SR)

    @pl.when(cid == 0)
    def _():
      run(C0_CH, sid * C0_CH)

    if C1_CH:
      @pl.when(cid == 1)
      def _():
        run(C1_CH, NS * C0_CH + sid * C1_CH)

    plsc.subcore_barrier()

    pltpu.sync_copy(acc_sh.at[pl.ds(base, RPT)],
                    out_s.at[cid, pl.ds(base, RPT)])

  return pl.kernel(body, out_type=out_type, mesh=_sc_mesh(),
                   scratch_types=scratch_types, name="sc_segsum")


@functools.lru_cache(maxsize=None)
def _make_sc_count():
  out_type = jax.ShapeDtypeStruct((NC, ACC_ROWS, D), jnp.float32)

  scratch_types = [
      pltpu.VMEM((NCH, CH), jnp.int32),
      pltpu.VMEM((CH, D), jnp.float32),
      pltpu.VMEM_SHARED((ACC_ROWS, D), jnp.float32),
  ]

  def body(dst_hbm, zcnt_hbm, ones_hbm, out_c, dst_v, ones_v, cnt_sh):
    cid = lax.axis_index("c")
    sid = lax.axis_index("s")
    wid = cid * NS + sid
    base = sid * RPT

    pltpu.sync_copy(dst_hbm.at[wid], dst_v)
    pltpu.sync_copy(zcnt_hbm, cnt_sh.at[pl.ds(base, RPT)])
    pltpu.sync_copy(ones_hbm, ones_v)
    plsc.subcore_barrier()

    @pl.loop(0, NCH)
    def _chunk(j):
      pltpu.sync_copy(ones_v, cnt_sh.at[dst_v.at[j]], add=True)

    plsc.subcore_barrier()
    pltpu.sync_copy(cnt_sh.at[pl.ds(base, RPT)],
                    out_c.at[cid, pl.ds(base, RPT)])

  return pl.kernel(body, out_type=out_type, mesh=_sc_mesh(),
                   scratch_types=scratch_types, name="sc_count")


_DN = (((1,), (1,)), ((), ()))


def _tc_in_body(x_ref, wl_ref, wr_ref, bl_ref, u_ref, v_ref):
  h = x_ref[...]
  u_ref[...] = lax.dot_general(h, wl_ref[...], _DN,
                               preferred_element_type=jnp.float32)
  v_ref[...] = lax.dot_general(h, wr_ref[...], _DN,
                               preferred_element_type=jnp.float32) + bl_ref[...]


def _tc_mid_body(relu, s_ref, c_ref, vp_ref, wl_ref, wr_ref, bl_ref,
                 u_ref, v_ref):
  cnt = c_ref[0] + c_ref[1]
  inv = 1.0 / jnp.maximum(cnt, 1.0)
  h = (s_ref[0] + s_ref[1]) * inv + vp_ref[...]
  if relu:
    h = jnp.maximum(h, 0.0)
  u_ref[...] = lax.dot_general(h, wl_ref[...], _DN,
                               preferred_element_type=jnp.float32)
  v_ref[...] = lax.dot_general(h, wr_ref[...], _DN,
                               preferred_element_type=jnp.float32) + bl_ref[...]


def _tc_fin_body(s_ref, c_ref, vp_ref, o_ref):
  cnt = c_ref[0] + c_ref[1]
  inv = 1.0 / jnp.maximum(cnt, 1.0)
  o_ref[...] = (s_ref[0] + s_ref[1]) * inv + vp_ref[...]


_ROW_SPEC = pl.BlockSpec((ROW_BLK, D), lambda i: (i, 0))
_W_SPEC = pl.BlockSpec((D, D), lambda i: (0, 0))
_B_SPEC = pl.BlockSpec((1, D), lambda i: (0, 0))
_S_SPEC = pl.BlockSpec((NC, ROW_BLK, D), lambda i: (0, i, 0))
_C_SPEC = pl.BlockSpec((NC, ROW_BLK, D), lambda i: (0, i, 0))
_GRID = (N_NODES // ROW_BLK,)
_UV_OUT = [jax.ShapeDtypeStruct((N_NODES, D), jnp.float32)] * 2

_tc_in = pl.pallas_call(
    _tc_in_body, grid=_GRID,
    in_specs=[_ROW_SPEC, _W_SPEC, _W_SPEC, _B_SPEC],
    out_specs=[_ROW_SPEC, _ROW_SPEC], out_shape=_UV_OUT)

_tc_mid_relu = pl.pallas_call(
    functools.partial(_tc_mid_body, True), grid=_GRID,
    in_specs=[_S_SPEC, _C_SPEC, _ROW_SPEC, _W_SPEC, _W_SPEC, _B_SPEC],
    out_specs=[_ROW_SPEC, _ROW_SPEC], out_shape=_UV_OUT)

_tc_mid = pl.pallas_call(
    functools.partial(_tc_mid_body, False), grid=_GRID,
    in_specs=[_S_SPEC, _C_SPEC, _ROW_SPEC, _W_SPEC, _W_SPEC, _B_SPEC],
    out_specs=[_ROW_SPEC, _ROW_SPEC], out_shape=_UV_OUT)

_tc_fin = pl.pallas_call(
    _tc_fin_body, grid=_GRID,
    in_specs=[_S_SPEC, _C_SPEC, _ROW_SPEC],
    out_specs=_ROW_SPEC,
    out_shape=jax.ShapeDtypeStruct((N_NODES, D), jnp.float32))


def kernel(x, edge_index, Wl1, bl1, Wr1, Wl2, bl2, Wr2, Wl3, bl3, Wr3):
  src = edge_index[0].astype(jnp.int32)
  dst = edge_index[1].astype(jnp.int32)
  pad = E_PAD - src.shape[0]
  src_c = jnp.concatenate(
      [src, jnp.zeros((pad,), jnp.int32)]).reshape(TOT_CH, CH)
  dst_flat = jnp.concatenate([dst, jnp.full((pad,), TRASH_ROW, jnp.int32)])
  dst_c = dst_flat.reshape(TOT_CH, CH)
  dst_p = dst_flat.reshape(NW, NCH, CH)
  zrow = jnp.zeros((RPT, D), jnp.float32)
  ones = jnp.ones((CH, D), jnp.float32)
  bl1r, bl2r, bl3r = (b.reshape(1, D) for b in (bl1, bl2, bl3))

  sc_segsum = _make_sc_segsum()
  sc_count = _make_sc_count()

  c1 = sc_count(dst_p, zrow, ones)
  u1, v1 = _tc_in(x, Wl1, Wr1, bl1r)
  s1 = sc_segsum(u1, src_c, dst_c, zrow)
  u2, v2 = _tc_mid_relu(s1, c1, v1, Wl2, Wr2, bl2r)
  s2 = sc_segsum(u2, src_c, dst_c, zrow)
  u3, v3 = _tc_mid(s2, c1, v2, Wl3, Wr3, bl3r)
  s3 = sc_segsum(u3, src_c, dst_c, zrow)
  return _tc_fin(s3, c1, v3)


# speedup vs baseline: 1.3953x; 1.3953x over previous
import functools

import jax
import jax.numpy as jnp
from jax import lax
from jax.experimental import pallas as pl
from jax.experimental.pallas import tpu as pltpu
from jax.experimental.pallas import tpu_sc as plsc

N_NODES = 10000
N_EDGES = 320000
D = 128

NC = 2
NS = 16
NW = NC * NS
CH = 128
NCH = 80
EPT = NCH * CH
E_PAD = NW * EPT
ACC_ROWS = 10240
RPT = ACC_ROWS // NS
TRASH_ROW = N_NODES + 7
ROW_BLK = 400
TOT_CH = E_PAD // CH
C0_CH = 156
C1_CH = TOT_CH // NS - C0_CH


def _sc_mesh():
  return plsc.VectorSubcoreMesh(core_axis_name="c", subcore_axis_name="s",
                                num_cores=NC, num_subcores=NS)


@functools.lru_cache(maxsize=None)
def _make_sc_segsum():
  out_type = jax.ShapeDtypeStruct((NC, ACC_ROWS, D), jnp.float32)

  NBUF = 2
  SR = 4
  scratch_types = [
      pltpu.VMEM((SR, CH), jnp.int32),
      pltpu.VMEM((SR, CH), jnp.int32),
      pltpu.VMEM((NBUF * CH, D), jnp.float32),
      pltpu.VMEM_SHARED((ACC_ROWS, D), jnp.float32),
      pltpu.SemaphoreType.DMA((NBUF,)),
      pltpu.SemaphoreType.DMA((SR,)),
      pltpu.SemaphoreType.DMA((SR,)),
  ]

  def body(u_hbm, src_hbm, dst_hbm, zrow_hbm, out_s,
           src_r, dst_r, rows_v, acc_sh, gsem, ssem, dsem):
    rows = [rows_v.at[pl.ds(p * CH, CH)] for p in range(NBUF)]
    cid = lax.axis_index("c")
    sid = lax.axis_index("s")
    base = sid * RPT

    pltpu.sync_copy(zrow_hbm, acc_sh.at[pl.ds(base, RPT)])
    plsc.subcore_barrier()

    def run(nch, ck0):
      for r in range(SR):
        pltpu.async_copy(src_hbm.at[ck0 + r], src_r.at[r], ssem.at[r])
        pltpu.async_copy(dst_hbm.at[ck0 + r], dst_r.at[r], dsem.at[r])
      for p in range(NBUF):
        pltpu.make_async_copy(src_hbm.at[0], src_r.at[p], ssem.at[p]).wait()
        pltpu.async_copy(u_hbm.at[src_r.at[p]], rows[p], gsem.at[p])

      def step(j, b, p, pre_idx, pre_gather):
        pltpu.make_async_copy(u_hbm.at[pl.ds(0, CH)], rows[p],
                              gsem.at[p]).wait()
        pltpu.make_async_copy(dst_hbm.at[0], dst_r.at[b], dsem.at[b]).wait()
        pltpu.sync_copy(rows[p], acc_sh.at[dst_r.at[b]], add=True)
        if pre_idx:
          pltpu.async_copy(src_hbm.at[ck0 + j + SR], src_r.at[b], ssem.at[b])
          pltpu.async_copy(dst_hbm.at[ck0 + j + SR], dst_r.at[b], dsem.at[b])
        if pre_gather:
          b2 = b + NBUF if b + NBUF < SR else b + NBUF - SR
          pltpu.make_async_copy(src_hbm.at[0], src_r.at[b2],
                                ssem.at[b2]).wait()
          pltpu.async_copy(u_hbm.at[src_r.at[b2]], rows[p], gsem.at[p])

      @pl.loop(0, nch - SR, step=SR)
      def _chunk(j0):
        for b in range(SR):
          step(j0 + b, b, b % NBUF, True, True)

      for jj in range(SR):
        j = nch - SR + jj
        step(j, jj % SR, jj % NBUF, False, jj + NBUF < SR)

    @pl.when(cid == 0)
    def _():
      run(C0_CH, sid * C0_CH)

    if C1_CH:
      @pl.when(cid == 1)
      def _():
        run(C1_CH, NS * C0_CH + sid * C1_CH)

    plsc.subcore_barrier()

    pltpu.sync_copy(acc_sh.at[pl.ds(base, RPT)],
                    out_s.at[cid, pl.ds(base, RPT)])

  return pl.kernel(body, out_type=out_type, mesh=_sc_mesh(),
                   scratch_types=scratch_types, name="sc_segsum")


@functools.lru_cache(maxsize=None)
def _make_sc_count():
  out_type = jax.ShapeDtypeStruct((NC, ACC_ROWS, D), jnp.float32)

  scratch_types = [
      pltpu.VMEM((NCH, CH), jnp.int32),
      pltpu.VMEM((CH, D), jnp.float32),
      pltpu.VMEM_SHARED((ACC_ROWS, D), jnp.float32),
  ]

  def body(dst_hbm, zcnt_hbm, ones_hbm, out_c, dst_v, ones_v, cnt_sh):
    cid = lax.axis_index("c")
    sid = lax.axis_index("s")
    wid = cid * NS + sid
    base = sid * RPT

    pltpu.sync_copy(dst_hbm.at[wid], dst_v)
    pltpu.sync_copy(zcnt_hbm, cnt_sh.at[pl.ds(base, RPT)])
    pltpu.sync_copy(ones_hbm, ones_v)
    plsc.subcore_barrier()

    @pl.loop(0, NCH)
    def _chunk(j):
      pltpu.sync_copy(ones_v, cnt_sh.at[dst_v.at[j]], add=True)

    plsc.subcore_barrier()
    pltpu.sync_copy(cnt_sh.at[pl.ds(base, RPT)],
                    out_c.at[cid, pl.ds(base, RPT)])

  return pl.kernel(body, out_type=out_type, mesh=_sc_mesh(),
                   scratch_types=scratch_types, name="sc_count")


_DN = (((1,), (1,)), ((), ()))


def _tc_in_body(x_ref, wl_ref, wr_ref, bl_ref, u_ref, v_ref):
  h = x_ref[...]
  u_ref[...] = lax.dot_general(h, wl_ref[...], _DN,
                               preferred_element_type=jnp.float32)
  v_ref[...] = lax.dot_general(h, wr_ref[...], _DN,
                               preferred_element_type=jnp.float32) + bl_ref[...]


def _tc_mid_body(relu, s_ref, c_ref, vp_ref, wl_ref, wr_ref, bl_ref,
                 u_ref, v_ref):
  cnt = c_ref[0] + c_ref[1]
  inv = 1.0 / jnp.maximum(cnt, 1.0)
  h = (s_ref[0] + s_ref[1]) * inv + vp_ref[...]
  if relu:
    h = jnp.maximum(h, 0.0)
  u_ref[...] = lax.dot_general(h, wl_ref[...], _DN,
                               preferred_element_type=jnp.float32)
  v_ref[...] = lax.dot_general(h, wr_ref[...], _DN,
                               preferred_element_type=jnp.float32) + bl_ref[...]


def _tc_fin_body(s_ref, c_ref, vp_ref, o_ref):
  cnt = c_ref[0] + c_ref[1]
  inv = 1.0 / jnp.maximum(cnt, 1.0)
  o_ref[...] = (s_ref[0] + s_ref[1]) * inv + vp_ref[...]


_ROW_SPEC = pl.BlockSpec((ROW_BLK, D), lambda i: (i, 0))
_W_SPEC = pl.BlockSpec((D, D), lambda i: (0, 0))
_B_SPEC = pl.BlockSpec((1, D), lambda i: (0, 0))
_S_SPEC = pl.BlockSpec((NC, ROW_BLK, D), lambda i: (0, i, 0))
_C_SPEC = pl.BlockSpec((NC, ROW_BLK, D), lambda i: (0, i, 0))
_GRID = (N_NODES // ROW_BLK,)
_UV_OUT = [jax.ShapeDtypeStruct((N_NODES, D), jnp.float32)] * 2

_tc_in = pl.pallas_call(
    _tc_in_body, grid=_GRID,
    in_specs=[_ROW_SPEC, _W_SPEC, _W_SPEC, _B_SPEC],
    out_specs=[_ROW_SPEC, _ROW_SPEC], out_shape=_UV_OUT)

_tc_mid_relu = pl.pallas_call(
    functools.partial(_tc_mid_body, True), grid=_GRID,
    in_specs=[_S_SPEC, _C_SPEC, _ROW_SPEC, _W_SPEC, _W_SPEC, _B_SPEC],
    out_specs=[_ROW_SPEC, _ROW_SPEC], out_shape=_UV_OUT)

_tc_mid = pl.pallas_call(
    functools.partial(_tc_mid_body, False), grid=_GRID,
    in_specs=[_S_SPEC, _C_SPEC, _ROW_SPEC, _W_SPEC, _W_SPEC, _B_SPEC],
    out_specs=[_ROW_SPEC, _ROW_SPEC], out_shape=_UV_OUT)

_tc_fin = pl.pallas_call(
    _tc_fin_body, grid=_GRID,
    in_specs=[_S_SPEC, _C_SPEC, _ROW_SPEC],
    out_specs=_ROW_SPEC,
    out_shape=jax.ShapeDtypeStruct((N_NODES, D), jnp.float32))


def kernel(x, edge_index, Wl1, bl1, Wr1, Wl2, bl2, Wr2, Wl3, bl3, Wr3):
  src = edge_index[0].astype(jnp.int32)
  dst = edge_index[1].astype(jnp.int32)
  pad = E_PAD - src.shape[0]
  src_c = jnp.concatenate(
      [src, jnp.zeros((pad,), jnp.int32)]).reshape(TOT_CH, CH)
  dst_flat = jnp.concatenate([dst, jnp.full((pad,), TRASH_ROW, jnp.int32)])
  dst_c = dst_flat.reshape(TOT_CH, CH)
  dst_p = dst_flat.reshape(NW, NCH, CH)
  zrow = jnp.zeros((RPT, D), jnp.float32)
  ones = jnp.ones((CH, D), jnp.float32)
  bl1r, bl2r, bl3r = (b.reshape(1, D) for b in (bl1, bl2, bl3))

  sc_segsum = _make_sc_segsum()
  sc_count = _make_sc_count()

  c1 = sc_count(dst_p, zrow, ones)
  u1, v1 = _tc_in(x, Wl1, Wr1, bl1r)
  s1 = sc_segsum(u1, src_c, dst_c, zrow)
  u2, v2 = _tc_mid_relu(s1, c1, v1, Wl2, Wr2, bl2r)
  s2 = sc_segsum(u2, src_c, dst_c, zrow)
  u3, v3 = _tc_mid(s2, c1, v2, Wl3, Wr3, bl3r)
  s3 = sc_segsum(u3, src_c, dst_c, zrow)
  return _tc_fin(s3, c1, v3)
